# baseline (speedup 1.0000x reference)
import functools

import jax
import jax.numpy as jnp
from jax import lax
from jax.experimental import pallas as pl
from jax.experimental.pallas import tpu as pltpu
from jax.experimental.pallas import tpu_sc as plsc

Q = 1024
N = 100000
D = 32
K_NN = 1000
C = 10

CW = 3584
NCHUNK = 28
NPAD = CW * NCHUNK

QBITS = 20
SHIFT = QBITS + 4 - 13
HBINS = 1 << 13
QMAX = float((1 << QBITS) - 1)
BIGKEY = ((1 << QBITS) - 1) * 16 + 15

NTILES = 32
NG = 8
QG = Q // NG
QPT = QG // NTILES
NV = NPAD // 16
UNROLL = 16


def _tc_body(x_ref, tft_ref, y_ref, keys_ref, mmin_ref, mmax_ref):
    j = pl.program_id(0)
    last = jnp.int32(NCHUNK - 1)
    xm2 = x_ref[...] * -2.0
    tft = tft_ref[...]
    t2 = jnp.sum(tft * tft, axis=0, keepdims=True)
    d2 = jnp.dot(xm2, tft, preferred_element_type=jnp.float32) + t2
    col = lax.broadcasted_iota(jnp.int32, (1, CW), 1)
    valid = col < N - (NPAD - CW)

    @pl.when(j == 0)
    def _():
        mmin_ref[...] = jnp.full((QG, 128), 1e30, jnp.float32)
        mmax_ref[...] = jnp.full((QG, 128), -1e30, jnp.float32)

    @pl.when(j < last)
    def _():
        dmin = jnp.min(d2, axis=1, keepdims=True)
        dmax = jnp.max(d2, axis=1, keepdims=True)
        mmin_ref[...] = jnp.minimum(mmin_ref[...],
                                    jnp.broadcast_to(dmin, (QG, 128)))
        mmax_ref[...] = jnp.maximum(mmax_ref[...],
                                    jnp.broadcast_to(dmax, (QG, 128)))

    @pl.when(j == last)
    def _():
        dmin = jnp.min(jnp.where(valid, d2, 1e30), axis=1, keepdims=True)
        dmax = jnp.max(jnp.where(valid, d2, -1e30), axis=1, keepdims=True)
        mmin_ref[...] = jnp.minimum(mmin_ref[...],
                                    jnp.broadcast_to(dmin, (QG, 128)))
        mmax_ref[...] = jnp.maximum(mmax_ref[...],
                                    jnp.broadcast_to(dmax, (QG, 128)))

    @pl.when(j >= NCHUNK)
    def _():
        lo = mmin_ref[:, 0:1]
        hi = mmax_ref[:, 0:1]
        s = QMAX / jnp.maximum(hi - lo, 1e-20)
        b = -lo * s
        qv = jnp.minimum(d2 * s + b, QMAX)
        key = qv.astype(jnp.int32) * 16 + jnp.broadcast_to(y_ref[...], (QG, CW))

        @pl.when(j < 2 * NCHUNK - 1)
        def _():
            keys_ref[...] = key

        @pl.when(j == 2 * NCHUNK - 1)
        def _():
            keys_ref[...] = jnp.where(valid, key, jnp.int32(BIGKEY))


HALF = NPAD // 2
NVH = HALF // 16


def _sc_body(keys_hbm, out_hbm, row_v, hist_v, obuf_v, dsema, dsemb):
    cid = lax.axis_index("c")
    sid = lax.axis_index("s")
    wid = sid * 2 + cid
    q0 = wid * QPT

    zeros_i = jnp.zeros((16,), jnp.int32)
    ones_i = jnp.ones((16,), jnp.int32)
    kk = jnp.int32(K_NN)

    def half_copy(qq, h, sem):
        return pltpu.make_async_copy(
            keys_hbm.at[qq, pl.ds(h * HALF, HALF)],
            row_v.at[pl.ds(h * HALF, HALF)], sem)

    def zero_hist():
        @plsc.parallel_loop(0, HBINS // 16, 1, unroll=UNROLL)
        def _(b):
            hist_v[pl.ds(b * 16, 16)] = zeros_i

    zero_hist()
    half_copy(q0, 0, dsema).start()
    half_copy(q0, 1, dsemb).start()

    def per_query(i, _):
        q = q0 + i

        def p1(lo, hi):
            @plsc.parallel_loop(lo, hi, 1, unroll=UNROLL)
            def _(v):
                k = row_v[pl.ds(v * 16, 16)]
                plsc.addupdate_scatter(hist_v, [jnp.right_shift(k, SHIFT)],
                                       ones_i)

        half_copy(q, 0, dsema).wait()
        p1(0, NVH)

        @pl.when(i < QPT - 1)
        def _():
            half_copy(q + 1, 0, dsema).start()

        half_copy(q, 1, dsemb).wait()
        p1(NVH, NV)

        @pl.when(i < QPT - 1)
        def _():
            half_copy(q + 1, 1, dsemb).start()

        def blk(b, carry):
            run, bsel, cumb = carry
            acc = zeros_i
            for t in range(16):
                acc = acc + hist_v[pl.ds((b * 16 + t) * 16, 16)]
            btot = jnp.sum(acc)
            hit = (bsel < 0) & (run + btot >= kk)
            return (run + btot,
                    jnp.where(hit, b, bsel),
                    jnp.where(hit, run, cumb))
        _, bsel, cumb = lax.fori_loop(
            0, HBINS // 256, blk,
            (jnp.int32(0), jnp.int32(-1), jnp.int32(0)))

        def vs(t, carry):
            run, vsel, cumv = carry
            vec = hist_v[pl.ds((bsel * 16 + t) * 16, 16)]
            vtot = jnp.sum(vec)
            hit = (vsel < 0) & (run + vtot >= kk)
            return (run + vtot,
                    jnp.where(hit, t, vsel),
                    jnp.where(hit, run, cumv))
        _, vsel, cumv = lax.fori_loop(
            0, 16, vs, (cumb, jnp.int32(-1), jnp.int32(0)))

        vec = hist_v[pl.ds((bsel * 16 + vsel) * 16, 16)]
        cum = cumv + plsc.cumsum(vec)
        pos = plsc.all_reduce_ffs(cum >= kk)
        tau = (bsel * 256 + vsel * 16) + pos
        bound = jnp.left_shift(tau + 1, SHIFT)

        zero_hist()
        obuf_v[pl.ds(0, 16)] = bound
        pltpu.sync_copy(obuf_v, out_hbm.at[q])
        return 0

    lax.fori_loop(0, QPT, per_query, 0)


@functools.cache
def _sc_kernel():
    return pl.kernel(
        _sc_body,
        out_type=jax.ShapeDtypeStruct((QG, 16), jnp.int32),
        mesh=plsc.VectorSubcoreMesh(core_axis_name="c", subcore_axis_name="s",
                                    num_cores=2, num_subcores=16),
        compiler_params=pltpu.CompilerParams(needs_layout_passes=False),
        scratch_types=[
            pltpu.VMEM((NPAD,), jnp.int32),
            pltpu.VMEM((HBINS,), jnp.int32),
            pltpu.VMEM((16,), jnp.int32),
            pltpu.SemaphoreType.DMA,
            pltpu.SemaphoreType.DMA,
        ],
    )


def _tc_c_body(x_ref, tft_ref, y_ref, mmin_ref, mmax_ref, bnd_ref, out_ref):
    j = pl.program_id(0)
    xm2 = x_ref[...] * -2.0
    tft = tft_ref[...]
    t2 = jnp.sum(tft * tft, axis=0, keepdims=True)
    d2 = jnp.dot(xm2, tft, preferred_element_type=jnp.float32) + t2
    lo = mmin_ref[:, 0:1]
    hi = mmax_ref[:, 0:1]
    s = QMAX / jnp.maximum(hi - lo, 1e-20)
    b = -lo * s
    qv = jnp.minimum(d2 * s + b, QMAX)
    key = qv.astype(jnp.int32) * 16 + jnp.broadcast_to(y_ref[...], (QG, CW))
    bnd = bnd_ref[:, 0:1]
    below = key < bnd
    col = lax.broadcasted_iota(jnp.int32, (1, CW), 1)
    valid = col < N - (NPAD - CW)
    oh = (jnp.broadcast_to(y_ref[...], (16, CW))
          == lax.broadcasted_iota(jnp.int32, (16, CW), 0)
          ).astype(jnp.float32)

    def counts(m):
        return lax.dot_general(m.astype(jnp.float32), oh,
                               (((1,), (1,)), ((), ())),
                               preferred_element_type=jnp.float32)

    @pl.when(j == 0)
    def _():
        out_ref[...] = counts(below)

    @pl.when((j > 0) & (j < NCHUNK - 1))
    def _():
        out_ref[...] = out_ref[...] + counts(below)

    @pl.when(j == NCHUNK - 1)
    def _():
        acc = out_ref[...] + counts(below & valid)
        out_ref[...] = acc / jnp.sum(acc, axis=1, keepdims=True)


def kernel(x, train_feats, y_train):
    tft = jnp.zeros((D, NPAD), jnp.float32).at[:, :N].set(train_feats.T)
    ypad = jnp.full((1, NPAD), 15, jnp.int32).at[:, :N].set(
        y_train.astype(jnp.int32)[None, :])
    outs = []
    for g in range(NG):
        xg = lax.slice_in_dim(x, g * QG, (g + 1) * QG, axis=0)
        keys, mmin, mmax = pl.pallas_call(
            _tc_body,
            grid=(2 * NCHUNK,),
            in_specs=[
                pl.BlockSpec((QG, D), lambda j: (0, 0)),
                pl.BlockSpec(
                    (D, CW),
                    lambda j: (0, jnp.where(j >= NCHUNK, j - NCHUNK, j))),
                pl.BlockSpec(
                    (1, CW),
                    lambda j: (0, jnp.where(j >= NCHUNK, j - NCHUNK, j))),
            ],
            out_specs=[
                pl.BlockSpec((QG, CW),
                             lambda j: (0, jnp.maximum(j - NCHUNK, 0))),
                pl.BlockSpec((QG, 128), lambda j: (0, 0)),
                pl.BlockSpec((QG, 128), lambda j: (0, 0)),
            ],
            out_shape=[
                jax.ShapeDtypeStruct((QG, NPAD), jnp.int32),
                jax.ShapeDtypeStruct((QG, 128), jnp.float32),
                jax.ShapeDtypeStruct((QG, 128), jnp.float32),
            ],
        )(xg, tft, ypad)
        bnd = _sc_kernel()(keys)
        depth = pl.pallas_call(
            _tc_c_body,
            grid=(NCHUNK,),
            in_specs=[
                pl.BlockSpec((QG, D), lambda j: (0, 0)),
                pl.BlockSpec((D, CW), lambda j: (0, j)),
                pl.BlockSpec((1, CW), lambda j: (0, j)),
                pl.BlockSpec((QG, 128), lambda j: (0, 0)),
                pl.BlockSpec((QG, 128), lambda j: (0, 0)),
                pl.BlockSpec((QG, 16), lambda j: (0, 0)),
            ],
            out_specs=pl.BlockSpec((QG, 16), lambda j: (0, 0)),
            out_shape=jax.ShapeDtypeStruct((QG, 16), jnp.float32),
        )(xg, tft, ypad, mmin, mmax, bnd)
        outs.append(depth)
    return jnp.concatenate(outs, axis=0)[:, :C]


# speedup vs baseline: 1.3200x; 1.3200x over previous
import functools

import jax
import jax.numpy as jnp
from jax import lax
from jax.experimental import pallas as pl
from jax.experimental.pallas import tpu as pltpu
from jax.experimental.pallas import tpu_sc as plsc

Q = 1024
N = 100000
D = 32
K_NN = 1000
C = 10

CW = 3584
NCHUNK = 28
NPAD = CW * NCHUNK

QBITS = 20
SHIFT = QBITS + 4 - 13
HBINS = 1 << 13
QMAX = float((1 << QBITS) - 1)
BIGKEY = ((1 << QBITS) - 1) * 16 + 15

NTILES = 32
NG = 4
QG = Q // NG
QPT = QG // NTILES
NV = NPAD // 16
UNROLL = 16


def _tc_body(x_ref, tft_ref, y_ref, keys_ref, mmin_ref, mmax_ref):
    j = pl.program_id(0)
    last = jnp.int32(NCHUNK - 1)
    xm2 = x_ref[...] * -2.0
    tft = tft_ref[...]
    t2 = jnp.sum(tft * tft, axis=0, keepdims=True)
    d2 = jnp.dot(xm2, tft, preferred_element_type=jnp.float32) + t2
    col = lax.broadcasted_iota(jnp.int32, (1, CW), 1)
    valid = col < N - (NPAD - CW)

    @pl.when(j == 0)
    def _():
        mmin_ref[...] = jnp.full((QG, 128), 1e30, jnp.float32)
        mmax_ref[...] = jnp.full((QG, 128), -1e30, jnp.float32)

    @pl.when(j < last)
    def _():
        dmin = jnp.min(d2, axis=1, keepdims=True)
        dmax = jnp.max(d2, axis=1, keepdims=True)
        mmin_ref[...] = jnp.minimum(mmin_ref[...],
                                    jnp.broadcast_to(dmin, (QG, 128)))
        mmax_ref[...] = jnp.maximum(mmax_ref[...],
                                    jnp.broadcast_to(dmax, (QG, 128)))

    @pl.when(j == last)
    def _():
        dmin = jnp.min(jnp.where(valid, d2, 1e30), axis=1, keepdims=True)
        dmax = jnp.max(jnp.where(valid, d2, -1e30), axis=1, keepdims=True)
        mmin_ref[...] = jnp.minimum(mmin_ref[...],
                                    jnp.broadcast_to(dmin, (QG, 128)))
        mmax_ref[...] = jnp.maximum(mmax_ref[...],
                                    jnp.broadcast_to(dmax, (QG, 128)))

    @pl.when(j >= NCHUNK)
    def _():
        lo = mmin_ref[:, 0:1]
        hi = mmax_ref[:, 0:1]
        s = QMAX / jnp.maximum(hi - lo, 1e-20)
        b = -lo * s
        qv = jnp.minimum(d2 * s + b, QMAX)
        key = qv.astype(jnp.int32) * 16 + jnp.broadcast_to(y_ref[...], (QG, CW))

        @pl.when(j < 2 * NCHUNK - 1)
        def _():
            keys_ref[...] = key

        @pl.when(j == 2 * NCHUNK - 1)
        def _():
            keys_ref[...] = jnp.where(valid, key, jnp.int32(BIGKEY))


HALF = NPAD // 2
NVH = HALF // 16


def _sc_body(keys_hbm, out_hbm, row_v, hist_v, obuf_v, dsema, dsemb):
    cid = lax.axis_index("c")
    sid = lax.axis_index("s")
    wid = sid * 2 + cid
    q0 = wid * QPT

    zeros_i = jnp.zeros((16,), jnp.int32)
    ones_i = jnp.ones((16,), jnp.int32)
    kk = jnp.int32(K_NN)

    def half_copy(qq, h, sem):
        return pltpu.make_async_copy(
            keys_hbm.at[qq, pl.ds(h * HALF, HALF)],
            row_v.at[pl.ds(h * HALF, HALF)], sem)

    def zero_hist():
        @plsc.parallel_loop(0, HBINS // 16, 1, unroll=UNROLL)
        def _(b):
            hist_v[pl.ds(b * 16, 16)] = zeros_i

    zero_hist()
    half_copy(q0, 0, dsema).start()
    half_copy(q0, 1, dsemb).start()

    def per_query(i, _):
        q = q0 + i

        def p1(lo, hi):
            @plsc.parallel_loop(lo, hi, 1, unroll=UNROLL)
            def _(v):
                k = row_v[pl.ds(v * 16, 16)]
                plsc.addupdate_scatter(hist_v, [jnp.right_shift(k, SHIFT)],
                                       ones_i)

        half_copy(q, 0, dsema).wait()
        p1(0, NVH)

        @pl.when(i < QPT - 1)
        def _():
            half_copy(q + 1, 0, dsema).start()

        half_copy(q, 1, dsemb).wait()
        p1(NVH, NV)

        @pl.when(i < QPT - 1)
        def _():
            half_copy(q + 1, 1, dsemb).start()

        def blk(b, carry):
            run, bsel, cumb = carry
            acc = zeros_i
            for t in range(16):
                acc = acc + hist_v[pl.ds((b * 16 + t) * 16, 16)]
            btot = jnp.sum(acc)
            hit = (bsel < 0) & (run + btot >= kk)
            return (run + btot,
                    jnp.where(hit, b, bsel),
                    jnp.where(hit, run, cumb))
        _, bsel, cumb = lax.fori_loop(
            0, HBINS // 256, blk,
            (jnp.int32(0), jnp.int32(-1), jnp.int32(0)))

        def vs(t, carry):
            run, vsel, cumv = carry
            vec = hist_v[pl.ds((bsel * 16 + t) * 16, 16)]
            vtot = jnp.sum(vec)
            hit = (vsel < 0) & (run + vtot >= kk)
            return (run + vtot,
                    jnp.where(hit, t, vsel),
                    jnp.where(hit, run, cumv))
        _, vsel, cumv = lax.fori_loop(
            0, 16, vs, (cumb, jnp.int32(-1), jnp.int32(0)))

        vec = hist_v[pl.ds((bsel * 16 + vsel) * 16, 16)]
        cum = cumv + plsc.cumsum(vec)
        pos = plsc.all_reduce_ffs(cum >= kk)
        tau = (bsel * 256 + vsel * 16) + pos
        bound = jnp.left_shift(tau + 1, SHIFT)

        zero_hist()
        obuf_v[pl.ds(0, 16)] = bound
        pltpu.sync_copy(obuf_v, out_hbm.at[q])
        return 0

    lax.fori_loop(0, QPT, per_query, 0)


@functools.cache
def _sc_kernel():
    return pl.kernel(
        _sc_body,
        out_type=jax.ShapeDtypeStruct((QG, 16), jnp.int32),
        mesh=plsc.VectorSubcoreMesh(core_axis_name="c", subcore_axis_name="s",
                                    num_cores=2, num_subcores=16),
        compiler_params=pltpu.CompilerParams(needs_layout_passes=False),
        scratch_types=[
            pltpu.VMEM((NPAD,), jnp.int32),
            pltpu.VMEM((HBINS,), jnp.int32),
            pltpu.VMEM((16,), jnp.int32),
            pltpu.SemaphoreType.DMA,
            pltpu.SemaphoreType.DMA,
        ],
    )


def _tc_c_body(x_ref, tft_ref, y_ref, mmin_ref, mmax_ref, bnd_ref, out_ref):
    j = pl.program_id(0)
    xm2 = x_ref[...] * -2.0
    tft = tft_ref[...]
    t2 = jnp.sum(tft * tft, axis=0, keepdims=True)
    d2 = jnp.dot(xm2, tft, preferred_element_type=jnp.float32) + t2
    lo = mmin_ref[:, 0:1]
    hi = mmax_ref[:, 0:1]
    s = QMAX / jnp.maximum(hi - lo, 1e-20)
    b = -lo * s
    qv = jnp.minimum(d2 * s + b, QMAX)
    key = qv.astype(jnp.int32) * 16 + jnp.broadcast_to(y_ref[...], (QG, CW))
    bnd = bnd_ref[:, 0:1]
    below = key < bnd
    col = lax.broadcasted_iota(jnp.int32, (1, CW), 1)
    valid = col < N - (NPAD - CW)
    oh = (jnp.broadcast_to(y_ref[...], (16, CW))
          == lax.broadcasted_iota(jnp.int32, (16, CW), 0)
          ).astype(jnp.float32)

    def counts(m):
        return lax.dot_general(m.astype(jnp.float32), oh,
                               (((1,), (1,)), ((), ())),
                               preferred_element_type=jnp.float32)

    @pl.when(j == 0)
    def _():
        out_ref[...] = counts(below)

    @pl.when((j > 0) & (j < NCHUNK - 1))
    def _():
        out_ref[...] = out_ref[...] + counts(below)

    @pl.when(j == NCHUNK - 1)
    def _():
        acc = out_ref[...] + counts(below & valid)
        out_ref[...] = acc / jnp.sum(acc, axis=1, keepdims=True)


def kernel(x, train_feats, y_train):
    tft = jnp.zeros((D, NPAD), jnp.float32).at[:, :N].set(train_feats.T)
    ypad = jnp.full((1, NPAD), 15, jnp.int32).at[:, :N].set(
        y_train.astype(jnp.int32)[None, :])
    outs = []
    for g in range(NG):
        xg = lax.slice_in_dim(x, g * QG, (g + 1) * QG, axis=0)
        keys, mmin, mmax = pl.pallas_call(
            _tc_body,
            grid=(2 * NCHUNK,),
            in_specs=[
                pl.BlockSpec((QG, D), lambda j: (0, 0)),
                pl.BlockSpec(
                    (D, CW),
                    lambda j: (0, jnp.where(j >= NCHUNK, j - NCHUNK, j))),
                pl.BlockSpec(
                    (1, CW),
                    lambda j: (0, jnp.where(j >= NCHUNK, j - NCHUNK, j))),
            ],
            out_specs=[
                pl.BlockSpec((QG, CW),
                             lambda j: (0, jnp.maximum(j - NCHUNK, 0))),
                pl.BlockSpec((QG, 128), lambda j: (0, 0)),
                pl.BlockSpec((QG, 128), lambda j: (0, 0)),
            ],
            out_shape=[
                jax.ShapeDtypeStruct((QG, NPAD), jnp.int32),
                jax.ShapeDtypeStruct((QG, 128), jnp.float32),
                jax.ShapeDtypeStruct((QG, 128), jnp.float32),
            ],
        )(xg, tft, ypad)
        bnd = _sc_kernel()(keys)
        depth = pl.pallas_call(
            _tc_c_body,
            grid=(NCHUNK,),
            in_specs=[
                pl.BlockSpec((QG, D), lambda j: (0, 0)),
                pl.BlockSpec((D, CW), lambda j: (0, j)),
                pl.BlockSpec((1, CW), lambda j: (0, j)),
                pl.BlockSpec((QG, 128), lambda j: (0, 0)),
                pl.BlockSpec((QG, 128), lambda j: (0, 0)),
                pl.BlockSpec((QG, 16), lambda j: (0, 0)),
            ],
            out_specs=pl.BlockSpec((QG, 16), lambda j: (0, 0)),
            out_shape=jax.ShapeDtypeStruct((QG, 16), jnp.float32),
        )(xg, tft, ypad, mmin, mmax, bnd)
        outs.append(depth)
    return jnp.concatenate(outs, axis=0)[:, :C]
